# Initial kernel scaffold; baseline (speedup 1.0000x reference)
import jax
import jax.numpy as jnp
from jax.experimental import pallas as pl


def kernel(seq, W):
    raise NotImplementedError("write your pallas kernel here")


# speedup vs baseline: 4.9869x; 4.9869x over previous
import jax
import jax.numpy as jnp
from jax.experimental import pallas as pl

_BLK_S = 512


def _bcast_kernel(w_ref, out_ref):
    out_ref[...] = jnp.broadcast_to(w_ref[...][None], out_ref.shape)


def kernel(seq, W):
    N, S = seq.shape
    E = W.shape[1]
    return pl.pallas_call(
        _bcast_kernel,
        grid=(S // _BLK_S,),
        in_specs=[pl.BlockSpec((_BLK_S, E), lambda i: (i, 0))],
        out_specs=pl.BlockSpec((N, _BLK_S, E), lambda i: (0, i, 0)),
        out_shape=jax.ShapeDtypeStruct((N, S, E), W.dtype),
    )(W[:S])
